# baseline (speedup 1.0000x reference)
import jax
import jax.numpy as jnp
from jax.experimental import pallas as pl
from jax.experimental.pallas import tpu as pltpu


_NEG_SLOPE = 0.2


def _dinv_of(deg):
    return jnp.where(deg > 0.0, 1.0 / jnp.sqrt(deg), 0.0)


def _contract0(a, b):
    return jax.lax.dot_general(a, b, (((0,), (0,)), ((), ())),
                               preferred_element_type=jnp.float32)


def _subj_kernel(t_ref, x_ref, dg_ref, ag_ref, wc_ref, w1_ref, b1_ref,
                 w2_ref, b2_ref,
                 xop_ref, adj_ref, degb_ref, hs_ref):
    i = pl.program_id(0)
    n = t_ref.shape[1]
    tdim = t_ref.shape[2]
    m = ag_ref.shape[0]

    wsd_col = wc_ref[0:tdim, 0:1] - wc_ref[0:tdim, 1:2]
    wrd_row = (wc_ref[tdim:2 * tdim, 0:1] - wc_ref[tdim:2 * tdim, 1:2]).T

    t = t_ref[0]
    tT = t.T
    t_relu = jnp.maximum(t, 0.0)
    tT_relu = jnp.maximum(tT, 0.0)

    dv = jnp.sum(t_relu * wrd_row, axis=1, keepdims=True)
    du = jnp.sum(tT_relu * wsd_col, axis=0, keepdims=True)
    score = dv + du + dg_ref[0]
    A = jnp.where(score >= 0.0, 1.0, 0.0)
    adj_ref[0] = A

    rows = jax.lax.broadcasted_iota(jnp.int32, (n, n), 0)
    cols = jax.lax.broadcasted_iota(jnp.int32, (n, n), 1)
    eye = jnp.where(rows == cols, 1.0, 0.0)
    A_hat = jnp.maximum(A, eye)

    ones_n = jnp.ones((n, 1), jnp.float32)
    deg = _contract0(A_hat, ones_n)
    dinv = _dinv_of(deg)

    h0 = jnp.dot(x_ref[0], w1_ref[...], preferred_element_type=jnp.float32)
    z1 = _contract0(A_hat, dinv * h0)
    h = jnp.maximum(dinv * z1 + b1_ref[...], 0.0)
    hs2 = dinv * jnp.dot(h, w2_ref[...], preferred_element_type=jnp.float32)
    z2 = _contract0(A_hat, hs2)
    xop_ref[0] = dinv * z2 + b2_ref[...]

    srows = jax.lax.broadcasted_iota(jnp.int32, (m, n), 0)
    scols = jax.lax.broadcasted_iota(jnp.int32, (m, n), 1)
    eyeb = jnp.where(srows == i * n + scols, 1.0, 0.0).astype(jnp.bfloat16)
    aghat = jnp.maximum(ag_ref[...], eyeb)
    ones_m = jnp.ones((m, 1), jnp.bfloat16)
    degb = _contract0(aghat, ones_m)
    degb_ref[...] = jnp.broadcast_to(degb, (n, 8))
    hs_ref[...] = (_dinv_of(degb) * h0).astype(jnp.bfloat16)


def _subjects(t_b, x_b, dg, A_g, w_cat, w1, b1r, w2, b2r):
    bz, n, tdim = t_b.shape
    m = A_g.shape[0]
    f0 = x_b.shape[2]
    f1 = w1.shape[1]
    f2 = w2.shape[1]
    sub3 = lambda i: (i, 0, 0)
    wmap = lambda i: (0, 0)
    return pl.pallas_call(
        _subj_kernel,
        grid=(bz,),
        in_specs=[
            pl.BlockSpec((1, n, tdim), sub3),
            pl.BlockSpec((1, n, f0), sub3),
            pl.BlockSpec((1, n, n), sub3),
            pl.BlockSpec((m, n), lambda i: (0, i)),
            pl.BlockSpec((2 * tdim, 2), wmap),
            pl.BlockSpec((f0, f1), wmap),
            pl.BlockSpec((1, f1), wmap),
            pl.BlockSpec((f1, f2), wmap),
            pl.BlockSpec((1, f2), wmap),
        ],
        out_specs=(pl.BlockSpec((1, n, f2), sub3),
                   pl.BlockSpec((1, n, n), sub3),
                   pl.BlockSpec((n, 8), lambda i: (i, 0)),
                   pl.BlockSpec((n, f1), lambda i: (i, 0))),
        out_shape=(jax.ShapeDtypeStruct((bz, n, f2), jnp.float32),
                   jax.ShapeDtypeStruct((bz, n, n), jnp.float32),
                   jax.ShapeDtypeStruct((m, 8), jnp.float32),
                   jax.ShapeDtypeStruct((m, f1), jnp.bfloat16)),
        compiler_params=pltpu.CompilerParams(
            dimension_semantics=("parallel",)),
    )(t_b, x_b, dg, A_g, w_cat, w1, b1r, w2, b2r)


def _corr1_kernel(ag_ref, hs_ref, degb_ref, b1_ref, w2_ref, hs2_ref):
    i = pl.program_id(0)
    m = ag_ref.shape[0]
    hb = ag_ref.shape[1]

    srows = jax.lax.broadcasted_iota(jnp.int32, (m, hb), 0)
    scols = jax.lax.broadcasted_iota(jnp.int32, (m, hb), 1)
    eyeb = jnp.where(srows == i * hb + scols, 1.0, 0.0).astype(jnp.bfloat16)
    ahat = jnp.maximum(ag_ref[...], eyeb)

    z = _contract0(ahat, hs_ref[...])
    dinv_blk = _dinv_of(degb_ref[pl.ds(i * hb, hb), 0:1])
    h = jnp.maximum(dinv_blk * z + b1_ref[...], 0.0)
    hs2 = dinv_blk * jnp.dot(h, w2_ref[...],
                             preferred_element_type=jnp.float32)
    hs2_ref[...] = hs2.astype(jnp.bfloat16)


def _corr1(A_g, hs, degb, b1r, w2):
    m = A_g.shape[0]
    f1 = hs.shape[1]
    f2 = w2.shape[1]
    hb = m // 2
    return pl.pallas_call(
        _corr1_kernel,
        grid=(2,),
        in_specs=[
            pl.BlockSpec((m, hb), lambda i: (0, i)),
            pl.BlockSpec((m, f1), lambda i: (0, 0)),
            pl.BlockSpec((m, 8), lambda i: (0, 0)),
            pl.BlockSpec((1, f1), lambda i: (0, 0)),
            pl.BlockSpec((f1, f2), lambda i: (0, 0)),
        ],
        out_specs=pl.BlockSpec((hb, f2), lambda i: (i, 0)),
        out_shape=jax.ShapeDtypeStruct((m, f2), jnp.bfloat16),
        compiler_params=pltpu.CompilerParams(
            dimension_semantics=("parallel",)),
    )(A_g, hs, degb, b1r, w2)


def _corr2_kernel(ag_ref, degb_ref, hs2_ref, b2_ref, xc_ref):
    i = pl.program_id(0)
    m = ag_ref.shape[0]
    hb = ag_ref.shape[1]

    srows = jax.lax.broadcasted_iota(jnp.int32, (m, hb), 0)
    scols = jax.lax.broadcasted_iota(jnp.int32, (m, hb), 1)
    eyeb = jnp.where(srows == i * hb + scols, 1.0, 0.0).astype(jnp.bfloat16)
    ahat = jnp.maximum(ag_ref[...], eyeb)

    z = _contract0(ahat, hs2_ref[...])
    dinv_blk = _dinv_of(degb_ref[pl.ds(i * hb, hb), 0:1])
    xc_ref[...] = dinv_blk * z + b2_ref[...]


def _corr2(A_g, degb, hs2, b2r):
    m = A_g.shape[0]
    f2 = hs2.shape[1]
    hb = m // 2
    return pl.pallas_call(
        _corr2_kernel,
        grid=(2,),
        in_specs=[
            pl.BlockSpec((m, hb), lambda i: (0, i)),
            pl.BlockSpec((m, 8), lambda i: (0, 0)),
            pl.BlockSpec((m, f2), lambda i: (0, 0)),
            pl.BlockSpec((1, f2), lambda i: (0, 0)),
        ],
        out_specs=pl.BlockSpec((hb, f2), lambda i: (i, 0)),
        out_shape=jax.ShapeDtypeStruct((m, f2), jnp.float32),
        compiler_params=pltpu.CompilerParams(
            dimension_semantics=("parallel",)),
    )(A_g, degb, hs2, b2r)


def _head1_kernel(emb_ref, w3_ref, hp_ref):
    j = pl.program_id(1)
    acc = jnp.dot(emb_ref[...], w3_ref[...],
                  preferred_element_type=jnp.float32)

    @pl.when(j == 0)
    def _():
        hp_ref[...] = acc[None]

    @pl.when(j != 0)
    def _():
        hp_ref[...] += acc[None]


def _head1(emb, w3, kchunks_per_core=2):
    bz, ktot = emb.shape
    h3 = w3.shape[1]
    kc = kchunks_per_core
    chunk = ktot // (2 * kc)
    return pl.pallas_call(
        _head1_kernel,
        grid=(2, kc),
        in_specs=[
            pl.BlockSpec((bz, chunk), lambda i, j: (0, i * kc + j)),
            pl.BlockSpec((chunk, h3), lambda i, j: (i * kc + j, 0)),
        ],
        out_specs=pl.BlockSpec((1, bz, h3), lambda i, j: (i, 0, 0)),
        out_shape=jax.ShapeDtypeStruct((2, bz, h3), jnp.float32),
        compiler_params=pltpu.CompilerParams(
            dimension_semantics=("parallel", "arbitrary")),
    )(emb, w3)


def _head2_kernel(hp_ref, b3_ref, w4_ref, b4_ref, w5_ref, b5_ref, adj_ref,
                  o_ref, adj7_ref):
    h = hp_ref[0] + hp_ref[1] + b3_ref[...]
    h = jnp.where(h >= 0.0, h, _NEG_SLOPE * h)
    y = jnp.dot(h, w4_ref[...], preferred_element_type=jnp.float32) + b4_ref[...]
    y = jnp.where(y >= 0.0, y, _NEG_SLOPE * y)
    o_ref[...] = jnp.dot(y, w5_ref[...],
                         preferred_element_type=jnp.float32) + b5_ref[...]
    adj7_ref[...] = adj_ref[0]


def _head2(hp, b3r, w4, b4r, w5, b5r, adj_all):
    bz = hp.shape[1]
    c = w5.shape[1]
    n = adj_all.shape[1]
    spec2 = lambda s: pl.BlockSpec(s, lambda i: (0,) * len(s))
    return pl.pallas_call(
        _head2_kernel,
        grid=(1,),
        in_specs=[spec2(hp.shape), spec2(b3r.shape), spec2(w4.shape),
                  spec2(b4r.shape), spec2(w5.shape), spec2(b5r.shape),
                  pl.BlockSpec((1, n, n), lambda i: (adj_all.shape[0] - 1, 0, 0))],
        out_specs=(spec2((bz, c)), spec2((n, n))),
        out_shape=(jax.ShapeDtypeStruct((bz, c), jnp.float32),
                   jax.ShapeDtypeStruct((n, n), jnp.float32)),
    )(hp, b3r, w4, b4r, w5, b5r, adj_all)


def kernel(x, t, edge_index, rng_key, w1, b1, w2, b2, w_cat, b_cat,
           w3, b3, w4, b4, w5, b5):
    n = 128
    m_total, f0 = x.shape
    bz = m_total // n
    tdim = t.shape[1]
    f2 = w2.shape[1]

    key = jax.random.wrap_key_data(rng_key)
    g = jax.random.gumbel(key, (bz, n, n, 2), jnp.float32)
    db = b_cat[0] - b_cat[1]
    dg = g[..., 0] - g[..., 1] + db

    t_b = t.reshape(bz, n, tdim)
    x_b = x.reshape(bz, n, f0)
    b1r = b1.reshape(1, -1)
    b2r = b2.reshape(1, -1)

    A_g = jnp.zeros((m_total, m_total), jnp.bfloat16)
    A_g = A_g.at[edge_index[0], edge_index[1]].set(
        jnp.bfloat16(1.0), unique_indices=True)

    x_op, adj_all, degb, hs = _subjects(t_b, x_b, dg, A_g, w_cat,
                                        w1, b1r, w2, b2r)
    hs2 = _corr1(A_g, hs, degb, b1r, w2)
    x_corr = _corr2(A_g, degb, hs2, b2r)

    emb = jnp.concatenate([x_op, x_corr.reshape(bz, n, f2)],
                          axis=2).reshape(bz, 2 * f2 * n)
    hp = _head1(emb, w3)
    out, adj7 = _head2(hp, b3.reshape(1, -1), w4, b4.reshape(1, -1),
                       w5, b5.reshape(1, -1), adj_all)
    return out, edge_index, adj7


# speedup vs baseline: 1.2984x; 1.2984x over previous
import jax
import jax.numpy as jnp
from jax.experimental import pallas as pl
from jax.experimental.pallas import tpu as pltpu


_NEG_SLOPE = 0.2


def _dinv_of(deg):
    return jnp.where(deg > 0.0, 1.0 / jnp.sqrt(deg), 0.0)


def _contract0(a, b):
    return jax.lax.dot_general(a, b, (((0,), (0,)), ((), ())),
                               preferred_element_type=jnp.float32)


def _subj_kernel(t_ref, x_ref, dg_ref, ag_ref, wc_ref, w1_ref, b1_ref,
                 w2_ref, b2_ref,
                 xop_ref, adj_ref, degb_ref, hs_ref):
    i = pl.program_id(0)
    n = t_ref.shape[1]
    tdim = t_ref.shape[2]
    m = ag_ref.shape[0]

    wsd_col = wc_ref[0:tdim, 0:1] - wc_ref[0:tdim, 1:2]
    wrd_row = (wc_ref[tdim:2 * tdim, 0:1] - wc_ref[tdim:2 * tdim, 1:2]).T

    t = t_ref[0]
    tT = t.T
    t_relu = jnp.maximum(t, 0.0)
    tT_relu = jnp.maximum(tT, 0.0)

    dv = jnp.sum(t_relu * wrd_row, axis=1, keepdims=True)
    du = jnp.sum(tT_relu * wsd_col, axis=0, keepdims=True)
    score = dv + du + dg_ref[0]
    A = jnp.where(score >= 0.0, 1.0, 0.0)
    adj_ref[0] = A

    rows = jax.lax.broadcasted_iota(jnp.int32, (n, n), 0)
    cols = jax.lax.broadcasted_iota(jnp.int32, (n, n), 1)
    eye = jnp.where(rows == cols, 1.0, 0.0)
    A_hat = jnp.maximum(A, eye)

    ones_n = jnp.ones((n, 1), jnp.float32)
    deg = _contract0(A_hat, ones_n)
    dinv = _dinv_of(deg)

    h0 = jnp.dot(x_ref[0], w1_ref[...], preferred_element_type=jnp.float32)
    z1 = _contract0(A_hat, dinv * h0)
    h = jnp.maximum(dinv * z1 + b1_ref[...], 0.0)
    hs2 = dinv * jnp.dot(h, w2_ref[...], preferred_element_type=jnp.float32)
    z2 = _contract0(A_hat, hs2)
    xop_ref[0] = dinv * z2 + b2_ref[...]

    srows = jax.lax.broadcasted_iota(jnp.int32, (m, n), 0)
    scols = jax.lax.broadcasted_iota(jnp.int32, (m, n), 1)
    eyeb = jnp.where(srows == i * n + scols, 1.0, 0.0)
    aghat = jnp.maximum(ag_ref[...], eyeb).astype(jnp.bfloat16)
    ones_m = jnp.ones((m, 1), jnp.bfloat16)
    degb = _contract0(aghat, ones_m)
    degb_ref[...] = jnp.broadcast_to(degb, (n, 8))
    hs_ref[...] = (_dinv_of(degb) * h0).astype(jnp.bfloat16)


def _subjects(t_b, x_b, dg, A_g, w_cat, w1, b1r, w2, b2r):
    bz, n, tdim = t_b.shape
    m = A_g.shape[0]
    f0 = x_b.shape[2]
    f1 = w1.shape[1]
    f2 = w2.shape[1]
    sub3 = lambda i: (i, 0, 0)
    wmap = lambda i: (0, 0)
    return pl.pallas_call(
        _subj_kernel,
        grid=(bz,),
        in_specs=[
            pl.BlockSpec((1, n, tdim), sub3),
            pl.BlockSpec((1, n, f0), sub3),
            pl.BlockSpec((1, n, n), sub3),
            pl.BlockSpec((m, n), lambda i: (0, i)),
            pl.BlockSpec((2 * tdim, 2), wmap),
            pl.BlockSpec((f0, f1), wmap),
            pl.BlockSpec((1, f1), wmap),
            pl.BlockSpec((f1, f2), wmap),
            pl.BlockSpec((1, f2), wmap),
        ],
        out_specs=(pl.BlockSpec((1, n, f2), sub3),
                   pl.BlockSpec((1, n, n), sub3),
                   pl.BlockSpec((n, 8), lambda i: (i, 0)),
                   pl.BlockSpec((n, f1), lambda i: (i, 0))),
        out_shape=(jax.ShapeDtypeStruct((bz, n, f2), jnp.float32),
                   jax.ShapeDtypeStruct((bz, n, n), jnp.float32),
                   jax.ShapeDtypeStruct((m, 8), jnp.float32),
                   jax.ShapeDtypeStruct((m, f1), jnp.bfloat16)),
        compiler_params=pltpu.CompilerParams(
            dimension_semantics=("parallel",)),
    )(t_b, x_b, dg, A_g, w_cat, w1, b1r, w2, b2r)


def _corr1_kernel(ag_ref, hs_ref, degb_ref, b1_ref, w2_ref, hs2_ref):
    i = pl.program_id(0)
    m = ag_ref.shape[0]
    hb = ag_ref.shape[1]

    srows = jax.lax.broadcasted_iota(jnp.int32, (m, hb), 0)
    scols = jax.lax.broadcasted_iota(jnp.int32, (m, hb), 1)
    eyeb = jnp.where(srows == i * hb + scols, 1.0, 0.0)
    ahat = jnp.maximum(ag_ref[...], eyeb).astype(jnp.bfloat16)

    z = _contract0(ahat, hs_ref[...])
    dinv_blk = _dinv_of(degb_ref[pl.ds(i * hb, hb), 0:1])
    h = jnp.maximum(dinv_blk * z + b1_ref[...], 0.0)
    hs2 = dinv_blk * jnp.dot(h, w2_ref[...],
                             preferred_element_type=jnp.float32)
    hs2_ref[...] = hs2.astype(jnp.bfloat16)


def _corr1(A_g, hs, degb, b1r, w2):
    m = A_g.shape[0]
    f1 = hs.shape[1]
    f2 = w2.shape[1]
    hb = m // 2
    return pl.pallas_call(
        _corr1_kernel,
        grid=(2,),
        in_specs=[
            pl.BlockSpec((m, hb), lambda i: (0, i)),
            pl.BlockSpec((m, f1), lambda i: (0, 0)),
            pl.BlockSpec((m, 8), lambda i: (0, 0)),
            pl.BlockSpec((1, f1), lambda i: (0, 0)),
            pl.BlockSpec((f1, f2), lambda i: (0, 0)),
        ],
        out_specs=pl.BlockSpec((hb, f2), lambda i: (i, 0)),
        out_shape=jax.ShapeDtypeStruct((m, f2), jnp.bfloat16),
        compiler_params=pltpu.CompilerParams(
            dimension_semantics=("parallel",)),
    )(A_g, hs, degb, b1r, w2)


def _corr2_kernel(ag_ref, degb_ref, hs2_ref, b2_ref, xc_ref):
    i = pl.program_id(0)
    m = ag_ref.shape[0]
    hb = ag_ref.shape[1]

    srows = jax.lax.broadcasted_iota(jnp.int32, (m, hb), 0)
    scols = jax.lax.broadcasted_iota(jnp.int32, (m, hb), 1)
    eyeb = jnp.where(srows == i * hb + scols, 1.0, 0.0)
    ahat = jnp.maximum(ag_ref[...], eyeb).astype(jnp.bfloat16)

    z = _contract0(ahat, hs2_ref[...])
    dinv_blk = _dinv_of(degb_ref[pl.ds(i * hb, hb), 0:1])
    xc_ref[...] = dinv_blk * z + b2_ref[...]


def _corr2(A_g, degb, hs2, b2r):
    m = A_g.shape[0]
    f2 = hs2.shape[1]
    hb = m // 2
    return pl.pallas_call(
        _corr2_kernel,
        grid=(2,),
        in_specs=[
            pl.BlockSpec((m, hb), lambda i: (0, i)),
            pl.BlockSpec((m, 8), lambda i: (0, 0)),
            pl.BlockSpec((m, f2), lambda i: (0, 0)),
            pl.BlockSpec((1, f2), lambda i: (0, 0)),
        ],
        out_specs=pl.BlockSpec((hb, f2), lambda i: (i, 0)),
        out_shape=jax.ShapeDtypeStruct((m, f2), jnp.float32),
        compiler_params=pltpu.CompilerParams(
            dimension_semantics=("parallel",)),
    )(A_g, degb, hs2, b2r)


def _head1_kernel(emb_ref, w3_ref, hp_ref):
    j = pl.program_id(1)
    acc = jnp.dot(emb_ref[...], w3_ref[...],
                  preferred_element_type=jnp.float32)

    @pl.when(j == 0)
    def _():
        hp_ref[...] = acc[None]

    @pl.when(j != 0)
    def _():
        hp_ref[...] += acc[None]


def _head1(emb, w3, kchunks_per_core=2):
    bz, ktot = emb.shape
    h3 = w3.shape[1]
    kc = kchunks_per_core
    chunk = ktot // (2 * kc)
    return pl.pallas_call(
        _head1_kernel,
        grid=(2, kc),
        in_specs=[
            pl.BlockSpec((bz, chunk), lambda i, j: (0, i * kc + j)),
            pl.BlockSpec((chunk, h3), lambda i, j: (i * kc + j, 0)),
        ],
        out_specs=pl.BlockSpec((1, bz, h3), lambda i, j: (i, 0, 0)),
        out_shape=jax.ShapeDtypeStruct((2, bz, h3), jnp.float32),
        compiler_params=pltpu.CompilerParams(
            dimension_semantics=("parallel", "arbitrary")),
    )(emb, w3)


def _head2_kernel(hp_ref, b3_ref, w4_ref, b4_ref, w5_ref, b5_ref, adj_ref,
                  o_ref, adj7_ref):
    h = hp_ref[0] + hp_ref[1] + b3_ref[...]
    h = jnp.where(h >= 0.0, h, _NEG_SLOPE * h)
    y = jnp.dot(h, w4_ref[...], preferred_element_type=jnp.float32) + b4_ref[...]
    y = jnp.where(y >= 0.0, y, _NEG_SLOPE * y)
    o_ref[...] = jnp.dot(y, w5_ref[...],
                         preferred_element_type=jnp.float32) + b5_ref[...]
    adj7_ref[...] = adj_ref[0]


def _head2(hp, b3r, w4, b4r, w5, b5r, adj_all):
    bz = hp.shape[1]
    c = w5.shape[1]
    n = adj_all.shape[1]
    spec2 = lambda s: pl.BlockSpec(s, lambda i: (0,) * len(s))
    return pl.pallas_call(
        _head2_kernel,
        grid=(1,),
        in_specs=[spec2(hp.shape), spec2(b3r.shape), spec2(w4.shape),
                  spec2(b4r.shape), spec2(w5.shape), spec2(b5r.shape),
                  pl.BlockSpec((1, n, n), lambda i: (adj_all.shape[0] - 1, 0, 0))],
        out_specs=(spec2((bz, c)), spec2((n, n))),
        out_shape=(jax.ShapeDtypeStruct((bz, c), jnp.float32),
                   jax.ShapeDtypeStruct((n, n), jnp.float32)),
    )(hp, b3r, w4, b4r, w5, b5r, adj_all)


def kernel(x, t, edge_index, rng_key, w1, b1, w2, b2, w_cat, b_cat,
           w3, b3, w4, b4, w5, b5):
    n = 128
    m_total, f0 = x.shape
    bz = m_total // n
    tdim = t.shape[1]
    f2 = w2.shape[1]

    key = jax.random.wrap_key_data(rng_key)
    g = jax.random.gumbel(key, (bz, n, n, 2), jnp.float32)
    db = b_cat[0] - b_cat[1]
    dg = g[..., 0] - g[..., 1] + db

    t_b = t.reshape(bz, n, tdim)
    x_b = x.reshape(bz, n, f0)
    b1r = b1.reshape(1, -1)
    b2r = b2.reshape(1, -1)

    A_g = jnp.zeros((m_total, m_total), jnp.float32)
    A_g = A_g.at[edge_index[0], edge_index[1]].set(1.0, unique_indices=True)

    x_op, adj_all, degb, hs = _subjects(t_b, x_b, dg, A_g, w_cat,
                                        w1, b1r, w2, b2r)
    hs2 = _corr1(A_g, hs, degb, b1r, w2)
    x_corr = _corr2(A_g, degb, hs2, b2r)

    emb = jnp.concatenate([x_op, x_corr.reshape(bz, n, f2)],
                          axis=2).reshape(bz, 2 * f2 * n)
    hp = _head1(emb, w3)
    out, adj7 = _head2(hp, b3.reshape(1, -1), w4, b4.reshape(1, -1),
                       w5, b5.reshape(1, -1), adj_all)
    return out, edge_index, adj7


# speedup vs baseline: 1.3617x; 1.0488x over previous
import jax
import jax.numpy as jnp
from jax.experimental import pallas as pl
from jax.experimental.pallas import tpu as pltpu


_NEG_SLOPE = 0.2


def _dinv_of(deg):
    return jnp.where(deg > 0.0, 1.0 / jnp.sqrt(deg), 0.0)


def _contract0(a, b):
    return jax.lax.dot_general(a, b, (((0,), (0,)), ((), ())),
                               preferred_element_type=jnp.float32)


def _subj_kernel(t_ref, x_ref, dg_ref, ag_ref, wc_ref, w1_ref, b1_ref,
                 w2_ref, b2_ref,
                 xop_ref, adj_ref, degb_ref, hs_ref):
    ns = t_ref.shape[0]
    n = t_ref.shape[1]
    tdim = t_ref.shape[2]

    wsd_col = wc_ref[0:tdim, 0:1] - wc_ref[0:tdim, 1:2]
    wrd_row = (wc_ref[tdim:2 * tdim, 0:1] - wc_ref[tdim:2 * tdim, 1:2]).T

    rows = jax.lax.broadcasted_iota(jnp.int32, (n, n), 0)
    cols = jax.lax.broadcasted_iota(jnp.int32, (n, n), 1)
    eye = jnp.where(rows == cols, 1.0, 0.0)
    ones_n = jnp.ones((n, 1), jnp.float32)

    aghat = ag_ref[...].astype(jnp.bfloat16)
    ones_m = jnp.ones((ag_ref.shape[0], 1), jnp.bfloat16)
    degb = _contract0(aghat, ones_m)
    degb_ref[...] = jnp.broadcast_to(degb, (ns * n, 8))
    dinv_g = _dinv_of(degb)

    for s in range(ns):
        t = t_ref[s]
        tT = t.T
        t_relu = jnp.maximum(t, 0.0)
        tT_relu = jnp.maximum(tT, 0.0)

        dv = jnp.sum(t_relu * wrd_row, axis=1, keepdims=True)
        du = jnp.sum(tT_relu * wsd_col, axis=0, keepdims=True)
        score = dv + du + dg_ref[s]
        A = jnp.where(score >= 0.0, 1.0, 0.0)
        adj_ref[s] = A

        A_hat = jnp.maximum(A, eye)
        deg = _contract0(A_hat, ones_n)
        dinv = _dinv_of(deg)

        h0 = jnp.dot(x_ref[s], w1_ref[...],
                     preferred_element_type=jnp.float32)
        z1 = _contract0(A_hat, dinv * h0)
        h = jnp.maximum(dinv * z1 + b1_ref[...], 0.0)
        hs2 = dinv * jnp.dot(h, w2_ref[...],
                             preferred_element_type=jnp.float32)
        z2 = _contract0(A_hat, hs2)
        xop_ref[s] = dinv * z2 + b2_ref[...]

        hs_ref[s * n:(s + 1) * n, :] = (
            dinv_g[s * n:(s + 1) * n, :] * h0).astype(jnp.bfloat16)


def _subjects(t_b, x_b, dg, A_g, w_cat, w1, b1r, w2, b2r, steps=4):
    bz, n, tdim = t_b.shape
    m = A_g.shape[0]
    f0 = x_b.shape[2]
    f1 = w1.shape[1]
    f2 = w2.shape[1]
    ns = bz // steps
    sub3 = lambda i: (i, 0, 0)
    wmap = lambda i: (0, 0)
    return pl.pallas_call(
        _subj_kernel,
        grid=(steps,),
        in_specs=[
            pl.BlockSpec((ns, n, tdim), sub3),
            pl.BlockSpec((ns, n, f0), sub3),
            pl.BlockSpec((ns, n, n), sub3),
            pl.BlockSpec((m, ns * n), lambda i: (0, i)),
            pl.BlockSpec((2 * tdim, 2), wmap),
            pl.BlockSpec((f0, f1), wmap),
            pl.BlockSpec((1, f1), wmap),
            pl.BlockSpec((f1, f2), wmap),
            pl.BlockSpec((1, f2), wmap),
        ],
        out_specs=(pl.BlockSpec((ns, n, f2), sub3),
                   pl.BlockSpec((ns, n, n), sub3),
                   pl.BlockSpec((ns * n, 8), lambda i: (i, 0)),
                   pl.BlockSpec((ns * n, f1), lambda i: (i, 0))),
        out_shape=(jax.ShapeDtypeStruct((bz, n, f2), jnp.float32),
                   jax.ShapeDtypeStruct((bz, n, n), jnp.float32),
                   jax.ShapeDtypeStruct((m, 8), jnp.float32),
                   jax.ShapeDtypeStruct((m, f1), jnp.bfloat16)),
        compiler_params=pltpu.CompilerParams(
            dimension_semantics=("parallel",)),
    )(t_b, x_b, dg, A_g, w_cat, w1, b1r, w2, b2r)


def _corr_kernel(ag_ref, hs_ref, degb_ref, b1_ref, w2_ref, b2_ref, xop_ref,
                 emb_ref):
    i = pl.program_id(0)
    m = ag_ref.shape[0]
    hb = m // 2
    sb = emb_ref.shape[0]
    n = emb_ref.shape[1]
    f2 = w2_ref.shape[1]

    ahat = ag_ref[...].astype(jnp.bfloat16)
    dinv = _dinv_of(degb_ref[...][:, 0:1])

    z1 = _contract0(ahat, hs_ref[...])
    h = jnp.maximum(dinv * z1 + b1_ref[...], 0.0)
    hs2 = (dinv * jnp.dot(h, w2_ref[...],
                          preferred_element_type=jnp.float32))
    hs2 = hs2.astype(jnp.bfloat16)

    ablk = ag_ref[:, pl.ds(i * hb, hb)].astype(jnp.bfloat16)
    z2 = _contract0(ablk, hs2)
    dinv_blk = _dinv_of(degb_ref[pl.ds(i * hb, hb), 0:1])
    xc = dinv_blk * z2 + b2_ref[...]

    emb_ref[:, :, 0:f2] = xop_ref[...]
    emb_ref[:, :, f2:2 * f2] = xc.reshape(sb, n, f2)


def _corr(A_g, hs, degb, b1r, w2, b2r, x_op):
    m = A_g.shape[0]
    f1 = hs.shape[1]
    f2 = w2.shape[1]
    bz, n = x_op.shape[0], x_op.shape[1]
    sb = bz // 2
    return pl.pallas_call(
        _corr_kernel,
        grid=(2,),
        in_specs=[
            pl.BlockSpec((m, m), lambda i: (0, 0)),
            pl.BlockSpec((m, f1), lambda i: (0, 0)),
            pl.BlockSpec((m, 8), lambda i: (0, 0)),
            pl.BlockSpec((1, f1), lambda i: (0, 0)),
            pl.BlockSpec((f1, f2), lambda i: (0, 0)),
            pl.BlockSpec((1, f2), lambda i: (0, 0)),
            pl.BlockSpec((sb, n, f2), lambda i: (i, 0, 0)),
        ],
        out_specs=pl.BlockSpec((sb, n, 2 * f2), lambda i: (i, 0, 0)),
        out_shape=jax.ShapeDtypeStruct((bz, n, 2 * f2), jnp.float32),
        compiler_params=pltpu.CompilerParams(
            dimension_semantics=("parallel",)),
    )(A_g, hs, degb, b1r, w2, b2r, x_op)


def _head1_kernel(emb_ref, w3_ref, hp_ref):
    j = pl.program_id(1)
    acc = jnp.dot(emb_ref[...], w3_ref[...],
                  preferred_element_type=jnp.float32)

    @pl.when(j == 0)
    def _():
        hp_ref[...] = acc[None]

    @pl.when(j != 0)
    def _():
        hp_ref[...] += acc[None]


def _head1(emb, w3, kchunks_per_core=2):
    bz, ktot = emb.shape
    h3 = w3.shape[1]
    kc = kchunks_per_core
    chunk = ktot // (2 * kc)
    return pl.pallas_call(
        _head1_kernel,
        grid=(2, kc),
        in_specs=[
            pl.BlockSpec((bz, chunk), lambda i, j: (0, i * kc + j)),
            pl.BlockSpec((chunk, h3), lambda i, j: (i * kc + j, 0)),
        ],
        out_specs=pl.BlockSpec((1, bz, h3), lambda i, j: (i, 0, 0)),
        out_shape=jax.ShapeDtypeStruct((2, bz, h3), jnp.float32),
        compiler_params=pltpu.CompilerParams(
            dimension_semantics=("parallel", "arbitrary")),
    )(emb, w3)


def _head2_kernel(hp_ref, b3_ref, w4_ref, b4_ref, w5_ref, b5_ref, adj_ref,
                  o_ref, adj7_ref):
    h = hp_ref[0] + hp_ref[1] + b3_ref[...]
    h = jnp.where(h >= 0.0, h, _NEG_SLOPE * h)
    y = jnp.dot(h, w4_ref[...], preferred_element_type=jnp.float32) + b4_ref[...]
    y = jnp.where(y >= 0.0, y, _NEG_SLOPE * y)
    o_ref[...] = jnp.dot(y, w5_ref[...],
                         preferred_element_type=jnp.float32) + b5_ref[...]
    adj7_ref[...] = adj_ref[0]


def _head2(hp, b3r, w4, b4r, w5, b5r, adj_all):
    bz = hp.shape[1]
    c = w5.shape[1]
    n = adj_all.shape[1]
    spec2 = lambda s: pl.BlockSpec(s, lambda i: (0,) * len(s))
    return pl.pallas_call(
        _head2_kernel,
        grid=(1,),
        in_specs=[spec2(hp.shape), spec2(b3r.shape), spec2(w4.shape),
                  spec2(b4r.shape), spec2(w5.shape), spec2(b5r.shape),
                  pl.BlockSpec((1, n, n), lambda i: (adj_all.shape[0] - 1, 0, 0))],
        out_specs=(spec2((bz, c)), spec2((n, n))),
        out_shape=(jax.ShapeDtypeStruct((bz, c), jnp.float32),
                   jax.ShapeDtypeStruct((n, n), jnp.float32)),
    )(hp, b3r, w4, b4r, w5, b5r, adj_all)


def kernel(x, t, edge_index, rng_key, w1, b1, w2, b2, w_cat, b_cat,
           w3, b3, w4, b4, w5, b5):
    n = 128
    m_total, f0 = x.shape
    bz = m_total // n
    tdim = t.shape[1]
    f2 = w2.shape[1]

    key = jax.random.wrap_key_data(rng_key)
    g = jax.random.gumbel(key, (bz, n, n, 2), jnp.float32)
    db = b_cat[0] - b_cat[1]
    dg = g[..., 0] - g[..., 1] + db

    t_b = t.reshape(bz, n, tdim)
    x_b = x.reshape(bz, n, f0)
    b1r = b1.reshape(1, -1)
    b2r = b2.reshape(1, -1)

    ii = jax.lax.broadcasted_iota(jnp.int32, (m_total, m_total), 0)
    jj = jax.lax.broadcasted_iota(jnp.int32, (m_total, m_total), 1)
    A_hat_g = jnp.where(ii == jj, 1.0, 0.0)
    A_hat_g = A_hat_g.at[edge_index[0], edge_index[1]].set(
        1.0, unique_indices=True)

    x_op, adj_all, degb, hs = _subjects(t_b, x_b, dg, A_hat_g, w_cat,
                                        w1, b1r, w2, b2r)
    emb3 = _corr(A_hat_g, hs, degb, b1r, w2, b2r, x_op)

    hp = _head1(emb3.reshape(bz, 2 * f2 * n), w3)
    out, adj7 = _head2(hp, b3.reshape(1, -1), w4, b4.reshape(1, -1),
                       w5, b5.reshape(1, -1), adj_all)
    return out, edge_index, adj7


# speedup vs baseline: 1.5392x; 1.1303x over previous
import jax
import jax.numpy as jnp
from jax.experimental import pallas as pl
from jax.experimental.pallas import tpu as pltpu


_NEG_SLOPE = 0.2


def _dinv_of(deg):
    return jnp.where(deg > 0.0, 1.0 / jnp.sqrt(deg), 0.0)


_TINY32 = 1.1754943508222875e-38


def _rotl(x, r):
    return (x << jnp.uint32(r)) | (x >> jnp.uint32(32 - r))


def _threefry_bits(k1, k2, x1):
    ks0 = k1
    ks1 = k2
    ks2 = k1 ^ k2 ^ jnp.uint32(0x1BD11BDA)
    x0 = jnp.broadcast_to(ks0, x1.shape)
    x1 = x1 + ks1
    rot = ((13, 15, 26, 6), (17, 29, 16, 24))
    inj = ((ks1, ks2, 1), (ks2, ks0, 2), (ks0, ks1, 3),
           (ks1, ks2, 4), (ks2, ks0, 5))
    for rnd in range(5):
        for r in rot[rnd % 2]:
            x0 = x0 + x1
            x1 = _rotl(x1, r)
            x1 = x1 ^ x0
        a, b, c = inj[rnd]
        x0 = x0 + a
        x1 = x1 + b + jnp.uint32(c)
    return x0 ^ x1


def _gumbel_bits(k1, k2, counter):
    bits = _threefry_bits(k1, k2, counter)
    fb = (bits >> jnp.uint32(9)) | jnp.uint32(0x3F800000)
    floats = jax.lax.bitcast_convert_type(fb, jnp.float32) - jnp.float32(1.0)
    tiny = jnp.float32(_TINY32)
    u = jnp.maximum(tiny, floats * jnp.float32(1.0) + tiny)
    return -jnp.log(-jnp.log(u))


def _contract0(a, b):
    return jax.lax.dot_general(a, b, (((0,), (0,)), ((), ())),
                               preferred_element_type=jnp.float32)


def _subj_kernel(t_ref, x_ref, kr_ref, bc_ref, ag_ref, wc_ref, w1_ref,
                 b1_ref, w2_ref, b2_ref,
                 xop_ref, adj_ref, degb_ref, hs_ref, agb_ref):
    i = pl.program_id(0)
    ns = t_ref.shape[0]
    n = t_ref.shape[1]
    tdim = t_ref.shape[2]
    k1 = kr_ref[0, 0]
    k2 = kr_ref[0, 1]
    db = bc_ref[0, 0] - bc_ref[0, 1]

    wsd_col = wc_ref[0:tdim, 0:1] - wc_ref[0:tdim, 1:2]
    wrd_row = (wc_ref[tdim:2 * tdim, 0:1] - wc_ref[tdim:2 * tdim, 1:2]).T

    rows = jax.lax.broadcasted_iota(jnp.int32, (n, n), 0)
    cols = jax.lax.broadcasted_iota(jnp.int32, (n, n), 1)
    eye = jnp.where(rows == cols, 1.0, 0.0)
    ones_n = jnp.ones((n, 1), jnp.float32)

    aghat = ag_ref[...].astype(jnp.bfloat16)
    agb_ref[...] = aghat
    ones_m = jnp.ones((ag_ref.shape[0], 1), jnp.bfloat16)
    degb = _contract0(aghat, ones_m)
    degb_ref[...] = jnp.broadcast_to(degb, (ns * n, 8))
    dinv_g = _dinv_of(degb)

    pr = jax.lax.broadcasted_iota(jnp.int32, (n, n), 0) * (2 * n)
    pc = jax.lax.broadcasted_iota(jnp.int32, (n, n), 1) * 2
    pbase = pr + pc

    for s in range(ns):
        t = t_ref[s]
        tT = t.T
        t_relu = jnp.maximum(t, 0.0)
        tT_relu = jnp.maximum(tT, 0.0)

        su = i * ns + s
        cnt = (pbase + su * (2 * n * n)).astype(jnp.uint32)
        dg = (_gumbel_bits(k1, k2, cnt)
              - _gumbel_bits(k1, k2, cnt + jnp.uint32(1))) + db

        dv = jnp.sum(t_relu * wrd_row, axis=1, keepdims=True)
        du = jnp.sum(tT_relu * wsd_col, axis=0, keepdims=True)
        score = dv + du + dg
        A = jnp.where(score >= 0.0, 1.0, 0.0)
        adj_ref[s] = A

        A_hat = jnp.maximum(A, eye)
        deg = _contract0(A_hat, ones_n)
        dinv = _dinv_of(deg)

        h0 = jnp.dot(x_ref[s], w1_ref[...],
                     preferred_element_type=jnp.float32)
        z1 = _contract0(A_hat, dinv * h0)
        h = jnp.maximum(dinv * z1 + b1_ref[...], 0.0)
        hs2 = dinv * jnp.dot(h, w2_ref[...],
                             preferred_element_type=jnp.float32)
        z2 = _contract0(A_hat, hs2)
        xop_ref[s] = dinv * z2 + b2_ref[...]

        hs_ref[s * n:(s + 1) * n, :] = (
            dinv_g[s * n:(s + 1) * n, :] * h0).astype(jnp.bfloat16)


def _subjects(t_b, x_b, kr, bc, A_g, w_cat, w1, b1r, w2, b2r, steps=4):
    bz, n, tdim = t_b.shape
    m = A_g.shape[0]
    f0 = x_b.shape[2]
    f1 = w1.shape[1]
    f2 = w2.shape[1]
    ns = bz // steps
    sub3 = lambda i: (i, 0, 0)
    wmap = lambda i: (0, 0)
    return pl.pallas_call(
        _subj_kernel,
        grid=(steps,),
        in_specs=[
            pl.BlockSpec((ns, n, tdim), sub3),
            pl.BlockSpec((ns, n, f0), sub3),
            pl.BlockSpec((1, 2), wmap),
            pl.BlockSpec((1, 2), wmap),
            pl.BlockSpec((m, ns * n), lambda i: (0, i)),
            pl.BlockSpec((2 * tdim, 2), wmap),
            pl.BlockSpec((f0, f1), wmap),
            pl.BlockSpec((1, f1), wmap),
            pl.BlockSpec((f1, f2), wmap),
            pl.BlockSpec((1, f2), wmap),
        ],
        out_specs=(pl.BlockSpec((ns, n, f2), sub3),
                   pl.BlockSpec((ns, n, n), sub3),
                   pl.BlockSpec((ns * n, 8), lambda i: (i, 0)),
                   pl.BlockSpec((ns * n, f1), lambda i: (i, 0)),
                   pl.BlockSpec((m, ns * n), lambda i: (0, i))),
        out_shape=(jax.ShapeDtypeStruct((bz, n, f2), jnp.float32),
                   jax.ShapeDtypeStruct((bz, n, n), jnp.float32),
                   jax.ShapeDtypeStruct((m, 8), jnp.float32),
                   jax.ShapeDtypeStruct((m, f1), jnp.bfloat16),
                   jax.ShapeDtypeStruct((m, m), jnp.bfloat16)),
        compiler_params=pltpu.CompilerParams(
            dimension_semantics=("parallel",)),
    )(t_b, x_b, kr, bc, A_g, w_cat, w1, b1r, w2, b2r)


def _corr_kernel(ag_ref, hs_ref, degb_ref, b1_ref, w2_ref, b2_ref, xop_ref,
                 emb_ref):
    i = pl.program_id(0)
    m = ag_ref.shape[0]
    hb = m // 2
    sb = emb_ref.shape[0]
    n = emb_ref.shape[1]
    f2 = w2_ref.shape[1]

    ahat = ag_ref[...]
    dinv = _dinv_of(degb_ref[...][:, 0:1])

    z1 = _contract0(ahat, hs_ref[...])
    h = jnp.maximum(dinv * z1 + b1_ref[...], 0.0)
    hs2 = (dinv * jnp.dot(h, w2_ref[...],
                          preferred_element_type=jnp.float32))
    hs2 = hs2.astype(jnp.bfloat16)

    ablk = ag_ref[:, pl.ds(i * hb, hb)]
    z2 = _contract0(ablk, hs2)
    dinv_blk = _dinv_of(degb_ref[pl.ds(i * hb, hb), 0:1])
    xc = dinv_blk * z2 + b2_ref[...]

    emb_ref[:, :, 0:f2] = xop_ref[...]
    emb_ref[:, :, f2:2 * f2] = xc.reshape(sb, n, f2)


def _corr(A_g, hs, degb, b1r, w2, b2r, x_op):
    m = A_g.shape[0]
    f1 = hs.shape[1]
    f2 = w2.shape[1]
    bz, n = x_op.shape[0], x_op.shape[1]
    sb = bz // 2
    return pl.pallas_call(
        _corr_kernel,
        grid=(2,),
        in_specs=[
            pl.BlockSpec((m, m), lambda i: (0, 0)),
            pl.BlockSpec((m, f1), lambda i: (0, 0)),
            pl.BlockSpec((m, 8), lambda i: (0, 0)),
            pl.BlockSpec((1, f1), lambda i: (0, 0)),
            pl.BlockSpec((f1, f2), lambda i: (0, 0)),
            pl.BlockSpec((1, f2), lambda i: (0, 0)),
            pl.BlockSpec((sb, n, f2), lambda i: (i, 0, 0)),
        ],
        out_specs=pl.BlockSpec((sb, n, 2 * f2), lambda i: (i, 0, 0)),
        out_shape=jax.ShapeDtypeStruct((bz, n, 2 * f2), jnp.float32),
        compiler_params=pltpu.CompilerParams(
            dimension_semantics=("parallel",)),
    )(A_g, hs, degb, b1r, w2, b2r, x_op)


def _head1_kernel(emb_ref, w3_ref, hp_ref):
    j = pl.program_id(1)
    acc = jnp.dot(emb_ref[...], w3_ref[...],
                  preferred_element_type=jnp.float32)

    @pl.when(j == 0)
    def _():
        hp_ref[...] = acc[None]

    @pl.when(j != 0)
    def _():
        hp_ref[...] += acc[None]


def _head1(emb, w3, kchunks_per_core=2):
    bz, ktot = emb.shape
    h3 = w3.shape[1]
    kc = kchunks_per_core
    chunk = ktot // (2 * kc)
    return pl.pallas_call(
        _head1_kernel,
        grid=(2, kc),
        in_specs=[
            pl.BlockSpec((bz, chunk), lambda i, j: (0, i * kc + j)),
            pl.BlockSpec((chunk, h3), lambda i, j: (i * kc + j, 0)),
        ],
        out_specs=pl.BlockSpec((1, bz, h3), lambda i, j: (i, 0, 0)),
        out_shape=jax.ShapeDtypeStruct((2, bz, h3), jnp.float32),
        compiler_params=pltpu.CompilerParams(
            dimension_semantics=("parallel", "arbitrary")),
    )(emb, w3)


def _head2_kernel(hp_ref, b3_ref, w4_ref, b4_ref, w5_ref, b5_ref, adj_ref,
                  o_ref, adj7_ref):
    h = hp_ref[0] + hp_ref[1] + b3_ref[...]
    h = jnp.where(h >= 0.0, h, _NEG_SLOPE * h)
    y = jnp.dot(h, w4_ref[...], preferred_element_type=jnp.float32) + b4_ref[...]
    y = jnp.where(y >= 0.0, y, _NEG_SLOPE * y)
    o_ref[...] = jnp.dot(y, w5_ref[...],
                         preferred_element_type=jnp.float32) + b5_ref[...]
    adj7_ref[...] = adj_ref[0]


def _head2(hp, b3r, w4, b4r, w5, b5r, adj_all):
    bz = hp.shape[1]
    c = w5.shape[1]
    n = adj_all.shape[1]
    spec2 = lambda s: pl.BlockSpec(s, lambda i: (0,) * len(s))
    return pl.pallas_call(
        _head2_kernel,
        grid=(1,),
        in_specs=[spec2(hp.shape), spec2(b3r.shape), spec2(w4.shape),
                  spec2(b4r.shape), spec2(w5.shape), spec2(b5r.shape),
                  pl.BlockSpec((1, n, n), lambda i: (adj_all.shape[0] - 1, 0, 0))],
        out_specs=(spec2((bz, c)), spec2((n, n))),
        out_shape=(jax.ShapeDtypeStruct((bz, c), jnp.float32),
                   jax.ShapeDtypeStruct((n, n), jnp.float32)),
    )(hp, b3r, w4, b4r, w5, b5r, adj_all)


def kernel(x, t, edge_index, rng_key, w1, b1, w2, b2, w_cat, b_cat,
           w3, b3, w4, b4, w5, b5):
    n = 128
    m_total, f0 = x.shape
    bz = m_total // n
    tdim = t.shape[1]
    f2 = w2.shape[1]

    t_b = t.reshape(bz, n, tdim)
    x_b = x.reshape(bz, n, f0)
    b1r = b1.reshape(1, -1)
    b2r = b2.reshape(1, -1)

    ii = jax.lax.broadcasted_iota(jnp.int32, (m_total, m_total), 0)
    jj = jax.lax.broadcasted_iota(jnp.int32, (m_total, m_total), 1)
    A_hat_g = jnp.where(ii == jj, 1.0, 0.0)
    A_hat_g = A_hat_g.at[edge_index[0], edge_index[1]].set(
        1.0, unique_indices=True)

    x_op, adj_all, degb, hs, ag_bf = _subjects(
        t_b, x_b, rng_key.reshape(1, 2), b_cat.reshape(1, 2).astype(jnp.float32),
        A_hat_g, w_cat, w1, b1r, w2, b2r)
    emb3 = _corr(ag_bf, hs, degb, b1r, w2, b2r, x_op)

    hp = _head1(emb3.reshape(bz, 2 * f2 * n), w3)
    out, adj7 = _head2(hp, b3.reshape(1, -1), w4, b4.reshape(1, -1),
                       w5, b5.reshape(1, -1), adj_all)
    return out, edge_index, adj7


# speedup vs baseline: 1.5430x; 1.0025x over previous
import jax
import jax.numpy as jnp
from jax.experimental import pallas as pl
from jax.experimental.pallas import tpu as pltpu


_NEG_SLOPE = 0.2


def _dinv_of(deg):
    return jnp.where(deg > 0.0, 1.0 / jnp.sqrt(deg), 0.0)


_TINY32 = 1.1754943508222875e-38


def _rotl(x, r):
    return (x << jnp.uint32(r)) | (x >> jnp.uint32(32 - r))


def _threefry_bits(k1, k2, x1):
    ks0 = k1
    ks1 = k2
    ks2 = k1 ^ k2 ^ jnp.uint32(0x1BD11BDA)
    x0 = jnp.broadcast_to(ks0, x1.shape)
    x1 = x1 + ks1
    rot = ((13, 15, 26, 6), (17, 29, 16, 24))
    inj = ((ks1, ks2, 1), (ks2, ks0, 2), (ks0, ks1, 3),
           (ks1, ks2, 4), (ks2, ks0, 5))
    for rnd in range(5):
        for r in rot[rnd % 2]:
            x0 = x0 + x1
            x1 = _rotl(x1, r)
            x1 = x1 ^ x0
        a, b, c = inj[rnd]
        x0 = x0 + a
        x1 = x1 + b + jnp.uint32(c)
    return x0 ^ x1


def _gumbel_bits(k1, k2, counter):
    bits = _threefry_bits(k1, k2, counter)
    fb = (bits >> jnp.uint32(9)) | jnp.uint32(0x3F800000)
    floats = jax.lax.bitcast_convert_type(fb, jnp.float32) - jnp.float32(1.0)
    tiny = jnp.float32(_TINY32)
    u = jnp.maximum(tiny, floats * jnp.float32(1.0) + tiny)
    return -jnp.log(-jnp.log(u))


def _contract0(a, b):
    return jax.lax.dot_general(a, b, (((0,), (0,)), ((), ())),
                               preferred_element_type=jnp.float32)


def _subj_kernel(t_ref, x_ref, kr_ref, bc_ref, ag_ref, wc_ref, w1_ref,
                 b1_ref, w2_ref, b2_ref,
                 xop_ref, adj_ref, degb_ref, hs_ref, agb_ref):
    i = pl.program_id(0)
    ns = t_ref.shape[0]
    n = t_ref.shape[1]
    tdim = t_ref.shape[2]
    k1 = kr_ref[0, 0]
    k2 = kr_ref[0, 1]
    db = bc_ref[0, 0] - bc_ref[0, 1]

    wsd_col = wc_ref[0:tdim, 0:1] - wc_ref[0:tdim, 1:2]
    wrd_row = (wc_ref[tdim:2 * tdim, 0:1] - wc_ref[tdim:2 * tdim, 1:2]).T

    rows = jax.lax.broadcasted_iota(jnp.int32, (n, n), 0)
    cols = jax.lax.broadcasted_iota(jnp.int32, (n, n), 1)
    eye = jnp.where(rows == cols, 1.0, 0.0)

    aghat = ag_ref[...].astype(jnp.bfloat16)
    agb_ref[...] = aghat
    ones_m = jnp.ones((ag_ref.shape[0], 1), jnp.bfloat16)
    degb = _contract0(aghat, ones_m)
    degb_ref[...] = jnp.broadcast_to(degb, (ns * n, 8))
    dinv_g = _dinv_of(degb)

    pr = jax.lax.broadcasted_iota(jnp.int32, (n, n), 0) * (2 * n)
    pc = jax.lax.broadcasted_iota(jnp.int32, (n, n), 1) * 2
    pbase = pr + pc

    for s in range(ns):
        t = t_ref[s]
        tT = t.T
        t_relu = jnp.maximum(t, 0.0)
        tT_relu = jnp.maximum(tT, 0.0)

        su = i * ns + s
        cnt = (pbase + su * (2 * n * n)).astype(jnp.uint32)
        dg = (_gumbel_bits(k1, k2, cnt)
              - _gumbel_bits(k1, k2, cnt + jnp.uint32(1))) + db

        dv = jnp.sum(t_relu * wrd_row, axis=1, keepdims=True)
        du = jnp.sum(tT_relu * wsd_col, axis=0, keepdims=True)
        score = dv + du + dg
        A = jnp.where(score >= 0.0, 1.0, 0.0)
        adj_ref[s] = A

        A_hat = jnp.maximum(A, eye)
        A_bf = A_hat.astype(jnp.bfloat16)
        deg = _contract0(A_bf, jnp.ones((n, 1), jnp.bfloat16))
        dinv = _dinv_of(deg)

        h0 = jnp.dot(x_ref[s], w1_ref[...],
                     preferred_element_type=jnp.float32)
        z1 = _contract0(A_bf, (dinv * h0).astype(jnp.bfloat16))
        h = jnp.maximum(dinv * z1 + b1_ref[...], 0.0)
        hs2 = dinv * jnp.dot(h, w2_ref[...],
                             preferred_element_type=jnp.float32)
        z2 = _contract0(A_bf, hs2.astype(jnp.bfloat16))
        xop_ref[s] = dinv * z2 + b2_ref[...]

        hs_ref[s * n:(s + 1) * n, :] = (
            dinv_g[s * n:(s + 1) * n, :] * h0).astype(jnp.bfloat16)


def _subjects(t_b, x_b, kr, bc, A_g, w_cat, w1, b1r, w2, b2r, steps=4):
    bz, n, tdim = t_b.shape
    m = A_g.shape[0]
    f0 = x_b.shape[2]
    f1 = w1.shape[1]
    f2 = w2.shape[1]
    ns = bz // steps
    sub3 = lambda i: (i, 0, 0)
    wmap = lambda i: (0, 0)
    return pl.pallas_call(
        _subj_kernel,
        grid=(steps,),
        in_specs=[
            pl.BlockSpec((ns, n, tdim), sub3),
            pl.BlockSpec((ns, n, f0), sub3),
            pl.BlockSpec((1, 2), wmap),
            pl.BlockSpec((1, 2), wmap),
            pl.BlockSpec((m, ns * n), lambda i: (0, i)),
            pl.BlockSpec((2 * tdim, 2), wmap),
            pl.BlockSpec((f0, f1), wmap),
            pl.BlockSpec((1, f1), wmap),
            pl.BlockSpec((f1, f2), wmap),
            pl.BlockSpec((1, f2), wmap),
        ],
        out_specs=(pl.BlockSpec((ns, n, f2), sub3),
                   pl.BlockSpec((ns, n, n), sub3),
                   pl.BlockSpec((ns * n, 8), lambda i: (i, 0)),
                   pl.BlockSpec((ns * n, f1), lambda i: (i, 0)),
                   pl.BlockSpec((m, ns * n), lambda i: (0, i))),
        out_shape=(jax.ShapeDtypeStruct((bz, n, f2), jnp.float32),
                   jax.ShapeDtypeStruct((bz, n, n), jnp.float32),
                   jax.ShapeDtypeStruct((m, 8), jnp.float32),
                   jax.ShapeDtypeStruct((m, f1), jnp.bfloat16),
                   jax.ShapeDtypeStruct((m, m), jnp.bfloat16)),
        compiler_params=pltpu.CompilerParams(
            dimension_semantics=("parallel",)),
    )(t_b, x_b, kr, bc, A_g, w_cat, w1, b1r, w2, b2r)


def _corr_kernel(ag_ref, hs_ref, degb_ref, b1_ref, w2_ref, b2_ref, xop_ref,
                 emb_ref):
    i = pl.program_id(0)
    m = ag_ref.shape[0]
    hb = m // 2
    sb = emb_ref.shape[0]
    n = emb_ref.shape[1]
    f2 = w2_ref.shape[1]

    ahat = ag_ref[...]
    dinv = _dinv_of(degb_ref[...][:, 0:1])

    z1 = _contract0(ahat, hs_ref[...])
    h = jnp.maximum(dinv * z1 + b1_ref[...], 0.0)
    hs2 = (dinv * jnp.dot(h, w2_ref[...],
                          preferred_element_type=jnp.float32))
    hs2 = hs2.astype(jnp.bfloat16)

    ablk = ag_ref[:, pl.ds(i * hb, hb)]
    z2 = _contract0(ablk, hs2)
    dinv_blk = _dinv_of(degb_ref[pl.ds(i * hb, hb), 0:1])
    xc = dinv_blk * z2 + b2_ref[...]

    emb_ref[:, :, 0:f2] = xop_ref[...]
    emb_ref[:, :, f2:2 * f2] = xc.reshape(sb, n, f2)


def _corr(A_g, hs, degb, b1r, w2, b2r, x_op):
    m = A_g.shape[0]
    f1 = hs.shape[1]
    f2 = w2.shape[1]
    bz, n = x_op.shape[0], x_op.shape[1]
    sb = bz // 2
    return pl.pallas_call(
        _corr_kernel,
        grid=(2,),
        in_specs=[
            pl.BlockSpec((m, m), lambda i: (0, 0)),
            pl.BlockSpec((m, f1), lambda i: (0, 0)),
            pl.BlockSpec((m, 8), lambda i: (0, 0)),
            pl.BlockSpec((1, f1), lambda i: (0, 0)),
            pl.BlockSpec((f1, f2), lambda i: (0, 0)),
            pl.BlockSpec((1, f2), lambda i: (0, 0)),
            pl.BlockSpec((sb, n, f2), lambda i: (i, 0, 0)),
        ],
        out_specs=pl.BlockSpec((sb, n, 2 * f2), lambda i: (i, 0, 0)),
        out_shape=jax.ShapeDtypeStruct((bz, n, 2 * f2), jnp.float32),
        compiler_params=pltpu.CompilerParams(
            dimension_semantics=("parallel",)),
    )(A_g, hs, degb, b1r, w2, b2r, x_op)


def _head1_kernel(emb_ref, w3_ref, hp_ref):
    j = pl.program_id(1)
    acc = jnp.dot(emb_ref[...], w3_ref[...],
                  preferred_element_type=jnp.float32)

    @pl.when(j == 0)
    def _():
        hp_ref[...] = acc[None]

    @pl.when(j != 0)
    def _():
        hp_ref[...] += acc[None]


def _head1(emb, w3, kchunks_per_core=1):
    bz, ktot = emb.shape
    h3 = w3.shape[1]
    kc = kchunks_per_core
    chunk = ktot // (2 * kc)
    return pl.pallas_call(
        _head1_kernel,
        grid=(2, kc),
        in_specs=[
            pl.BlockSpec((bz, chunk), lambda i, j: (0, i * kc + j)),
            pl.BlockSpec((chunk, h3), lambda i, j: (i * kc + j, 0)),
        ],
        out_specs=pl.BlockSpec((1, bz, h3), lambda i, j: (i, 0, 0)),
        out_shape=jax.ShapeDtypeStruct((2, bz, h3), jnp.float32),
        compiler_params=pltpu.CompilerParams(
            dimension_semantics=("parallel", "arbitrary")),
    )(emb, w3)


def _head2_kernel(hp_ref, b3_ref, w4_ref, b4_ref, w5_ref, b5_ref, adj_ref,
                  o_ref, adj7_ref):
    h = hp_ref[0] + hp_ref[1] + b3_ref[...]
    h = jnp.where(h >= 0.0, h, _NEG_SLOPE * h)
    y = jnp.dot(h, w4_ref[...], preferred_element_type=jnp.float32) + b4_ref[...]
    y = jnp.where(y >= 0.0, y, _NEG_SLOPE * y)
    o_ref[...] = jnp.dot(y, w5_ref[...],
                         preferred_element_type=jnp.float32) + b5_ref[...]
    adj7_ref[...] = adj_ref[0]


def _head2(hp, b3r, w4, b4r, w5, b5r, adj_all):
    bz = hp.shape[1]
    c = w5.shape[1]
    n = adj_all.shape[1]
    spec2 = lambda s: pl.BlockSpec(s, lambda i: (0,) * len(s))
    return pl.pallas_call(
        _head2_kernel,
        grid=(1,),
        in_specs=[spec2(hp.shape), spec2(b3r.shape), spec2(w4.shape),
                  spec2(b4r.shape), spec2(w5.shape), spec2(b5r.shape),
                  pl.BlockSpec((1, n, n), lambda i: (adj_all.shape[0] - 1, 0, 0))],
        out_specs=(spec2((bz, c)), spec2((n, n))),
        out_shape=(jax.ShapeDtypeStruct((bz, c), jnp.float32),
                   jax.ShapeDtypeStruct((n, n), jnp.float32)),
    )(hp, b3r, w4, b4r, w5, b5r, adj_all)


def kernel(x, t, edge_index, rng_key, w1, b1, w2, b2, w_cat, b_cat,
           w3, b3, w4, b4, w5, b5):
    n = 128
    m_total, f0 = x.shape
    bz = m_total // n
    tdim = t.shape[1]
    f2 = w2.shape[1]

    t_b = t.reshape(bz, n, tdim)
    x_b = x.reshape(bz, n, f0)
    b1r = b1.reshape(1, -1)
    b2r = b2.reshape(1, -1)

    ii = jax.lax.broadcasted_iota(jnp.int32, (m_total, m_total), 0)
    jj = jax.lax.broadcasted_iota(jnp.int32, (m_total, m_total), 1)
    A_hat_g = jnp.where(ii == jj, 1.0, 0.0)
    A_hat_g = A_hat_g.at[edge_index[0], edge_index[1]].set(
        1.0, unique_indices=True)

    x_op, adj_all, degb, hs, ag_bf = _subjects(
        t_b, x_b, rng_key.reshape(1, 2), b_cat.reshape(1, 2).astype(jnp.float32),
        A_hat_g, w_cat, w1, b1r, w2, b2r)
    emb3 = _corr(ag_bf, hs, degb, b1r, w2, b2r, x_op)

    hp = _head1(emb3.reshape(bz, 2 * f2 * n), w3)
    out, adj7 = _head2(hp, b3.reshape(1, -1), w4, b4.reshape(1, -1),
                       w5, b5.reshape(1, -1), adj_all)
    return out, edge_index, adj7


# speedup vs baseline: 1.6786x; 1.0879x over previous
import jax
import jax.numpy as jnp
from jax.experimental import pallas as pl
from jax.experimental.pallas import tpu as pltpu


_NEG_SLOPE = 0.2


def _dinv_of(deg):
    return jnp.where(deg > 0.0, 1.0 / jnp.sqrt(deg), 0.0)


_TINY32 = 1.1754943508222875e-38


def _rotl(x, r):
    return (x << jnp.uint32(r)) | (x >> jnp.uint32(32 - r))


def _threefry_bits(k1, k2, x1):
    ks0 = k1
    ks1 = k2
    ks2 = k1 ^ k2 ^ jnp.uint32(0x1BD11BDA)
    x0 = jnp.broadcast_to(ks0, x1.shape)
    x1 = x1 + ks1
    rot = ((13, 15, 26, 6), (17, 29, 16, 24))
    inj = ((ks1, ks2, 1), (ks2, ks0, 2), (ks0, ks1, 3),
           (ks1, ks2, 4), (ks2, ks0, 5))
    for rnd in range(5):
        for r in rot[rnd % 2]:
            x0 = x0 + x1
            x1 = _rotl(x1, r)
            x1 = x1 ^ x0
        a, b, c = inj[rnd]
        x0 = x0 + a
        x1 = x1 + b + jnp.uint32(c)
    return x0 ^ x1


def _gumbel_bits(k1, k2, counter):
    bits = _threefry_bits(k1, k2, counter)
    fb = (bits >> jnp.uint32(9)) | jnp.uint32(0x3F800000)
    floats = jax.lax.bitcast_convert_type(fb, jnp.float32) - jnp.float32(1.0)
    tiny = jnp.float32(_TINY32)
    u = jnp.maximum(tiny, floats * jnp.float32(1.0) + tiny)
    return -jnp.log(-jnp.log(u))


def _contract0(a, b):
    return jax.lax.dot_general(a, b, (((0,), (0,)), ((), ())),
                               preferred_element_type=jnp.float32)


def _subj_kernel(t_ref, x_ref, kr_ref, bc_ref, ag_ref, wc_ref, w1_ref,
                 b1_ref, w2_ref, b2_ref,
                 xop_ref, adj_ref, degb_ref, hs_ref, agb_ref):
    i = pl.program_id(0)
    ns = t_ref.shape[0]
    n = t_ref.shape[1]
    tdim = t_ref.shape[2]
    k1 = kr_ref[0, 0]
    k2 = kr_ref[0, 1]
    db = bc_ref[0, 0] - bc_ref[0, 1]

    wsd_col = wc_ref[0:tdim, 0:1] - wc_ref[0:tdim, 1:2]
    wrd_row = (wc_ref[tdim:2 * tdim, 0:1] - wc_ref[tdim:2 * tdim, 1:2]).T

    rows = jax.lax.broadcasted_iota(jnp.int32, (n, n), 0)
    cols = jax.lax.broadcasted_iota(jnp.int32, (n, n), 1)
    eye = jnp.where(rows == cols, 1.0, 0.0)

    aghat = ag_ref[...].astype(jnp.bfloat16)
    agb_ref[...] = aghat
    ones_m = jnp.ones((ag_ref.shape[0], 1), jnp.bfloat16)
    degb = _contract0(aghat, ones_m)
    degb_ref[...] = jnp.broadcast_to(degb, (ns * n, 8))
    dinv_g = _dinv_of(degb)

    pr = jax.lax.broadcasted_iota(jnp.int32, (n, n), 0) * (2 * n)
    pc = jax.lax.broadcasted_iota(jnp.int32, (n, n), 1) * 2
    pbase = pr + pc

    for s in range(ns):
        t = t_ref[s]
        tT = t.T
        t_relu = jnp.maximum(t, 0.0)
        tT_relu = jnp.maximum(tT, 0.0)

        su = i * ns + s
        cnt = (pbase + su * (2 * n * n)).astype(jnp.uint32)
        dg = (_gumbel_bits(k1, k2, cnt)
              - _gumbel_bits(k1, k2, cnt + jnp.uint32(1))) + db

        dv = jnp.sum(t_relu * wrd_row, axis=1, keepdims=True)
        du = jnp.sum(tT_relu * wsd_col, axis=0, keepdims=True)
        score = dv + du + dg
        A = jnp.where(score >= 0.0, 1.0, 0.0)
        adj_ref[s] = A

        A_hat = jnp.maximum(A, eye)
        A_bf = A_hat.astype(jnp.bfloat16)
        deg = _contract0(A_bf, jnp.ones((n, 1), jnp.bfloat16))
        dinv = _dinv_of(deg)

        h0 = jnp.dot(x_ref[s], w1_ref[...],
                     preferred_element_type=jnp.float32)
        z1 = _contract0(A_bf, (dinv * h0).astype(jnp.bfloat16))
        h = jnp.maximum(dinv * z1 + b1_ref[...], 0.0)
        hs2 = dinv * jnp.dot(h, w2_ref[...],
                             preferred_element_type=jnp.float32)
        z2 = _contract0(A_bf, hs2.astype(jnp.bfloat16))
        xop_ref[s] = dinv * z2 + b2_ref[...]

        hs_ref[s * n:(s + 1) * n, :] = (
            dinv_g[s * n:(s + 1) * n, :] * h0).astype(jnp.bfloat16)


def _subjects(t_b, x_b, kr, bc, A_g, w_cat, w1, b1r, w2, b2r, steps=4):
    bz, n, tdim = t_b.shape
    m = A_g.shape[0]
    f0 = x_b.shape[2]
    f1 = w1.shape[1]
    f2 = w2.shape[1]
    ns = bz // steps
    sub3 = lambda i: (i, 0, 0)
    wmap = lambda i: (0, 0)
    return pl.pallas_call(
        _subj_kernel,
        grid=(steps,),
        in_specs=[
            pl.BlockSpec((ns, n, tdim), sub3),
            pl.BlockSpec((ns, n, f0), sub3),
            pl.BlockSpec((1, 2), wmap),
            pl.BlockSpec((1, 2), wmap),
            pl.BlockSpec((m, ns * n), lambda i: (0, i)),
            pl.BlockSpec((2 * tdim, 2), wmap),
            pl.BlockSpec((f0, f1), wmap),
            pl.BlockSpec((1, f1), wmap),
            pl.BlockSpec((f1, f2), wmap),
            pl.BlockSpec((1, f2), wmap),
        ],
        out_specs=(pl.BlockSpec((ns, n, f2), sub3),
                   pl.BlockSpec((ns, n, n), sub3),
                   pl.BlockSpec((ns * n, 8), lambda i: (i, 0)),
                   pl.BlockSpec((ns * n, f1), lambda i: (i, 0)),
                   pl.BlockSpec((m, ns * n), lambda i: (0, i))),
        out_shape=(jax.ShapeDtypeStruct((bz, n, f2), jnp.float32),
                   jax.ShapeDtypeStruct((bz, n, n), jnp.float32),
                   jax.ShapeDtypeStruct((m, 8), jnp.float32),
                   jax.ShapeDtypeStruct((m, f1), jnp.bfloat16),
                   jax.ShapeDtypeStruct((m, m), jnp.bfloat16)),
        compiler_params=pltpu.CompilerParams(
            dimension_semantics=("arbitrary",)),
    )(t_b, x_b, kr, bc, A_g, w_cat, w1, b1r, w2, b2r)


def _corr_kernel(ag_ref, hs_ref, degb_ref, b1_ref, w2_ref, b2_ref, xop_ref,
                 emb_ref):
    bz = xop_ref.shape[0]
    n = xop_ref.shape[1]
    f2 = w2_ref.shape[1]

    ahat = ag_ref[...]
    dinv = _dinv_of(degb_ref[...][:, 0:1])

    z1 = _contract0(ahat, hs_ref[...])
    h = jnp.maximum(dinv * z1 + b1_ref[...], 0.0)
    hs2 = (dinv * jnp.dot(h, w2_ref[...],
                          preferred_element_type=jnp.float32))
    hs2 = hs2.astype(jnp.bfloat16)

    z2 = _contract0(ahat, hs2)
    xc = dinv * z2 + b2_ref[...]

    emb = jnp.concatenate([xop_ref[...], xc.reshape(bz, n, f2)], axis=2)
    emb_ref[...] = emb.reshape(bz, 2 * f2 * n)


def _corr(A_g, hs, degb, b1r, w2, b2r, x_op):
    m = A_g.shape[0]
    f1 = hs.shape[1]
    f2 = w2.shape[1]
    bz, n = x_op.shape[0], x_op.shape[1]
    full = lambda shape: pl.BlockSpec(shape, lambda i: (0,) * len(shape))
    return pl.pallas_call(
        _corr_kernel,
        grid=(1,),
        in_specs=[
            full((m, m)),
            full((m, f1)),
            full((m, 8)),
            full((1, f1)),
            full((f1, f2)),
            full((1, f2)),
            full((bz, n, f2)),
        ],
        out_specs=full((bz, 2 * f2 * n)),
        out_shape=jax.ShapeDtypeStruct((bz, 2 * f2 * n), jnp.float32),
    )(A_g, hs, degb, b1r, w2, b2r, x_op)


def _head_kernel(emb_ref, w3_ref, b3_ref, w4_ref, b4_ref, w5_ref, b5_ref,
                 adj_ref, hp_ref, o_ref, adj7_ref):
    j = pl.program_id(0)
    kc = pl.num_programs(0)
    acc = jnp.dot(emb_ref[...], w3_ref[...],
                  preferred_element_type=jnp.float32)

    @pl.when(j == 0)
    def _():
        hp_ref[...] = acc

    @pl.when((j > 0) & (j < kc - 1))
    def _():
        hp_ref[...] += acc

    @pl.when(j == kc - 1)
    def _():
        h = hp_ref[...] + acc + b3_ref[...]
        h = jnp.where(h >= 0.0, h, _NEG_SLOPE * h)
        y = (jnp.dot(h, w4_ref[...], preferred_element_type=jnp.float32)
             + b4_ref[...])
        y = jnp.where(y >= 0.0, y, _NEG_SLOPE * y)
        o_ref[...] = (jnp.dot(y, w5_ref[...],
                              preferred_element_type=jnp.float32)
                      + b5_ref[...])
        adj7_ref[...] = adj_ref[0]


def _head(emb, w3, b3r, w4, b4r, w5, b5r, adj_all, kchunks=2):
    bz, ktot = emb.shape
    h3 = w3.shape[1]
    c = w5.shape[1]
    n = adj_all.shape[1]
    chunk = ktot // kchunks
    wmap = lambda j: (0, 0)
    _, out, adj7 = pl.pallas_call(
        _head_kernel,
        grid=(kchunks,),
        in_specs=[
            pl.BlockSpec((bz, chunk), lambda j: (0, j)),
            pl.BlockSpec((chunk, h3), lambda j: (j, 0)),
            pl.BlockSpec((1, h3), wmap),
            pl.BlockSpec(w4.shape, wmap),
            pl.BlockSpec((1, w4.shape[1]), wmap),
            pl.BlockSpec(w5.shape, wmap),
            pl.BlockSpec((1, c), wmap),
            pl.BlockSpec((1, n, n), lambda j: (adj_all.shape[0] - 1, 0, 0)),
        ],
        out_specs=(pl.BlockSpec((bz, h3), wmap),
                   pl.BlockSpec((bz, c), wmap),
                   pl.BlockSpec((n, n), wmap)),
        out_shape=(jax.ShapeDtypeStruct((bz, h3), jnp.float32),
                   jax.ShapeDtypeStruct((bz, c), jnp.float32),
                   jax.ShapeDtypeStruct((n, n), jnp.float32)),
        compiler_params=pltpu.CompilerParams(
            dimension_semantics=("arbitrary",)),
    )(emb, w3, b3r, w4, b4r, w5, b5r, adj_all)
    return out, adj7


def kernel(x, t, edge_index, rng_key, w1, b1, w2, b2, w_cat, b_cat,
           w3, b3, w4, b4, w5, b5):
    n = 128
    m_total, f0 = x.shape
    bz = m_total // n
    tdim = t.shape[1]
    f2 = w2.shape[1]

    t_b = t.reshape(bz, n, tdim)
    x_b = x.reshape(bz, n, f0)
    b1r = b1.reshape(1, -1)
    b2r = b2.reshape(1, -1)

    ii = jax.lax.broadcasted_iota(jnp.int32, (m_total, m_total), 0)
    jj = jax.lax.broadcasted_iota(jnp.int32, (m_total, m_total), 1)
    A_hat_g = jnp.where(ii == jj, 1.0, 0.0)
    A_hat_g = A_hat_g.at[edge_index[0], edge_index[1]].set(
        1.0, unique_indices=True)

    x_op, adj_all, degb, hs, ag_bf = _subjects(
        t_b, x_b, rng_key.reshape(1, 2), b_cat.reshape(1, 2).astype(jnp.float32),
        A_hat_g, w_cat, w1, b1r, w2, b2r)
    emb = _corr(ag_bf, hs, degb, b1r, w2, b2r, x_op)

    out, adj7 = _head(emb, w3, b3.reshape(1, -1), w4, b4.reshape(1, -1),
                      w5, b5.reshape(1, -1), adj_all)
    return out, edge_index, adj7
